# Initial kernel scaffold; baseline (speedup 1.0000x reference)
import jax
import jax.numpy as jnp
from jax.experimental import pallas as pl


def kernel(edge_index, W1, al1, ar1, W2, al2, ar2, cw1, cb1, cw2, cb2, cw3, cb3, cw4, cb4, cw5, cb5):
    raise NotImplementedError("write your pallas kernel here")


# speedup vs baseline: 609.6793x; 609.6793x over previous
import functools

import jax
import jax.numpy as jnp
from jax import lax
from jax.experimental import pallas as pl
from jax.experimental.pallas import tpu as pltpu
from jax.experimental.pallas import tpu_sc as plsc

N = 50000
E = 800000
HID = 32
HEADS = 2
OUT = 10

NP = 51200
NSL = NP // 16
EP = 819200
TEDGE = EP // 32
CHUNK = 3200
NSUB = TEDGE // CHUNK
NPAD = EP - E


def _lk(x):
    return jnp.where(x > 0, x, x * jnp.float32(0.2))


def _perm(v, idx):
    dn = lax.GatherDimensionNumbers(offset_dims=(), collapsed_slice_dims=(0,),
                                    start_index_map=(0,))
    return lax.gather(v, idx[:, None], dn, (1,),
                      mode=lax.GatherScatterMode.PROMISE_IN_BOUNDS)


def _splat(vec, j):
    return _perm(vec, jnp.full((16,), j, jnp.int32))


def _lane_max_splat(v):
    iota = lax.iota(jnp.int32, 16)
    for k in (8, 4, 2, 1):
        v = jnp.maximum(v, _perm(v, lax.bitwise_xor(iota, k)))
    return v


def _zero1d(ref, n):
    def body(i, _):
        ref[pl.ds(i * 16, 16)] = jnp.zeros((16,), jnp.float32)
        return 0
    lax.fori_loop(0, n // 16, body, 0)


def _sc_mesh():
    return plsc.VectorSubcoreMesh(core_axis_name="c", subcore_axis_name="s")


def _pass0(srcp, dstp, valid):
    @functools.partial(
        pl.kernel, mesh=_sc_mesh(),
        out_type=jax.ShapeDtypeStruct((2, 1, NP), jnp.float32),
        scratch_types=[
            pltpu.VMEM((NSL,), jnp.float32),
            pltpu.VMEM((CHUNK,), jnp.int32),
            pltpu.VMEM((CHUNK,), jnp.float32),
            pltpu.VMEM_SHARED((NP,), jnp.float32),
        ],
    )
    def k(src_h, dst_h, val_h, out_h, zt, dst_v, val_v, deg_sp):
        c = lax.axis_index("c")
        s = lax.axis_index("s")
        wid = c * 16 + s
        row0 = s * NSL
        ebase = wid * TEDGE

        _zero1d(zt, NSL)
        pltpu.sync_copy(zt, deg_sp.at[pl.ds(row0, NSL)])
        plsc.subcore_barrier()

        def ebody(kk, _):
            b = ebase + kk * CHUNK
            pltpu.sync_copy(dst_h.at[pl.ds(b, CHUNK)], dst_v)
            pltpu.sync_copy(val_h.at[pl.ds(b, CHUNK)], val_v)
            pltpu.sync_copy(val_v, deg_sp.at[dst_v], add=True)
            return 0
        lax.fori_loop(0, NSUB, ebody, 0)
        plsc.subcore_barrier()

        pltpu.sync_copy(deg_sp.at[pl.ds(row0, NSL)], zt)
        pltpu.sync_copy(zt, out_h.at[c, 0, pl.ds(row0, NSL)])

    return k(srcp, dstp, valid)


def _pass1(srcp, dstp, valid, deg_p, params1):
    @functools.partial(
        pl.kernel, mesh=_sc_mesh(),
        out_type=[jax.ShapeDtypeStruct((2, 4, 1, NP), jnp.float32),
                  jax.ShapeDtypeStruct((2, 1, 16), jnp.float32)],
        scratch_types=[
            pltpu.VMEM((NSL,), jnp.float32),
            pltpu.VMEM((NSL,), jnp.float32),
            pltpu.VMEM((16,), jnp.float32),
            pltpu.VMEM((16,), jnp.float32),
            pltpu.VMEM((256,), jnp.float32),
            pltpu.VMEM((CHUNK,), jnp.int32),
            pltpu.VMEM((CHUNK,), jnp.int32),
            pltpu.VMEM((CHUNK,), jnp.float32),
            pltpu.VMEM((CHUNK,), jnp.float32),
            pltpu.VMEM((CHUNK,), jnp.float32),
            pltpu.VMEM((CHUNK,), jnp.float32),
            pltpu.VMEM((CHUNK,), jnp.float32),
            pltpu.VMEM((CHUNK,), jnp.float32),
            pltpu.VMEM((CHUNK,), jnp.float32),
            pltpu.VMEM_SHARED((NP,), jnp.float32),
            pltpu.VMEM_SHARED((NP,), jnp.float32),
            pltpu.VMEM_SHARED((NP,), jnp.float32),
            pltpu.VMEM_SHARED((NP,), jnp.float32),
            pltpu.VMEM_SHARED((NP,), jnp.float32),
            pltpu.VMEM_SHARED((256,), jnp.float32),
        ],
    )
    def k(src_h, dst_h, val_h, deg_h, par_h, acc_out, dmax_out,
          ta, tb, pv, tmpf, sbuf, src_v, dst_v, val_v, degs_v, degd_v,
          val0, val1, val2, val3, deg_sp, d0_sp, d1_sp, n0_sp, n1_sp,
          stats_sp):
        c = lax.axis_index("c")
        s = lax.axis_index("s")
        wid = c * 16 + s
        row0 = s * NSL
        ebase = wid * TEDGE

        _zero1d(ta, NSL)
        pltpu.sync_copy(ta, d0_sp.at[pl.ds(row0, NSL)])
        pltpu.sync_copy(ta, d1_sp.at[pl.ds(row0, NSL)])
        pltpu.sync_copy(ta, n0_sp.at[pl.ds(row0, NSL)])
        pltpu.sync_copy(ta, n1_sp.at[pl.ds(row0, NSL)])

        pltpu.sync_copy(deg_h.at[0, 0, pl.ds(row0, NSL)], ta)
        pltpu.sync_copy(deg_h.at[1, 0, pl.ds(row0, NSL)], tb)

        def stg(i, m):
            v = ta[pl.ds(i * 16, 16)] + tb[pl.ds(i * 16, 16)]
            ta[pl.ds(i * 16, 16)] = v
            return jnp.maximum(m, v)
        mvec = lax.fori_loop(0, NSL // 16, stg, jnp.zeros((16,), jnp.float32))
        pltpu.sync_copy(ta, deg_sp.at[pl.ds(row0, NSL)])
        tmpf[...] = mvec
        pltpu.sync_copy(tmpf, stats_sp.at[pl.ds(s * 16, 16)])
        pltpu.sync_copy(par_h, pv)
        plsc.subcore_barrier()

        pltpu.sync_copy(stats_sp, sbuf)
        m = sbuf[pl.ds(0, 16)]
        for i in range(1, 16):
            m = jnp.maximum(m, sbuf[pl.ds(i * 16, 16)])
        dmax = _lane_max_splat(m)

        pvv = pv[...]
        cl0 = _splat(pvv, 0); cl1 = _splat(pvv, 1)
        cr0 = _splat(pvv, 2); cr1 = _splat(pvv, 3)
        em0 = jnp.maximum(cl0, 0.0) * dmax
        em1 = jnp.maximum(cl1, 0.0) * dmax

        def ebody(kk, _):
            b = ebase + kk * CHUNK
            pltpu.sync_copy(src_h.at[pl.ds(b, CHUNK)], src_v)
            pltpu.sync_copy(dst_h.at[pl.ds(b, CHUNK)], dst_v)
            pltpu.sync_copy(val_h.at[pl.ds(b, CHUNK)], val_v)
            pltpu.sync_copy(deg_sp.at[src_v], degs_v)
            pltpu.sync_copy(deg_sp.at[dst_v], degd_v)

            def cbody(i, _):
                sl = pl.ds(i * 16, 16)
                ds_ = degs_v[sl]; dd = degd_v[sl]; vv = val_v[sl]
                er = dd * cr0
                ee0 = jnp.exp(_lk(ds_ * cl0 + er) - _lk(er + em0)) * vv
                er1 = dd * cr1
                ee1 = jnp.exp(_lk(ds_ * cl1 + er1) - _lk(er1 + em1)) * vv
                val0[sl] = ee0
                val1[sl] = ee1
                val2[sl] = ee0 * ds_
                val3[sl] = ee1 * ds_
                return 0
            lax.fori_loop(0, CHUNK // 16, cbody, 0)
            pltpu.sync_copy(val0, d0_sp.at[dst_v], add=True)
            pltpu.sync_copy(val1, d1_sp.at[dst_v], add=True)
            pltpu.sync_copy(val2, n0_sp.at[dst_v], add=True)
            pltpu.sync_copy(val3, n1_sp.at[dst_v], add=True)
            return 0
        lax.fori_loop(0, NSUB, ebody, 0)
        plsc.subcore_barrier()

        for comp, ref in enumerate([d0_sp, d1_sp, n0_sp, n1_sp]):
            pltpu.sync_copy(ref.at[pl.ds(row0, NSL)], ta)
            pltpu.sync_copy(ta, acc_out.at[c, comp, 0, pl.ds(row0, NSL)])

        @pl.when(s == 0)
        def _():
            tmpf[...] = dmax
            pltpu.sync_copy(tmpf, dmax_out.at[c, 0, :])

    return k(srcp, dstp, valid, deg_p, params1)


def _pass2(srcp, dstp, valid, acc1, dvec, params2):
    @functools.partial(
        pl.kernel, mesh=_sc_mesh(),
        out_type=jax.ShapeDtypeStruct((2, 6, 1, NP), jnp.float32),
        scratch_types=[
            pltpu.VMEM((NSL,), jnp.float32),
            pltpu.VMEM((NSL,), jnp.float32),
            pltpu.VMEM((NSL,), jnp.float32),
            pltpu.VMEM((16,), jnp.float32),
            pltpu.VMEM((16,), jnp.float32),
            pltpu.VMEM((16,), jnp.float32),
            pltpu.VMEM((CHUNK,), jnp.int32),
            pltpu.VMEM((CHUNK,), jnp.int32),
            pltpu.VMEM((CHUNK,), jnp.float32),
            pltpu.VMEM((CHUNK,), jnp.float32),
            pltpu.VMEM((CHUNK,), jnp.float32),
            pltpu.VMEM((CHUNK,), jnp.float32),
            pltpu.VMEM((CHUNK,), jnp.float32),
            pltpu.VMEM((CHUNK,), jnp.float32),
            pltpu.VMEM((CHUNK,), jnp.float32),
            pltpu.VMEM((CHUNK,), jnp.float32),
            pltpu.VMEM((CHUNK,), jnp.float32),
            pltpu.VMEM((CHUNK,), jnp.float32),
            pltpu.VMEM((CHUNK,), jnp.float32),
            pltpu.VMEM_SHARED((NP,), jnp.float32),
            pltpu.VMEM_SHARED((NP,), jnp.float32),
            pltpu.VMEM_SHARED((NP,), jnp.float32),
            pltpu.VMEM_SHARED((NP,), jnp.float32),
            pltpu.VMEM_SHARED((NP,), jnp.float32),
            pltpu.VMEM_SHARED((NP,), jnp.float32),
            pltpu.VMEM_SHARED((NP,), jnp.float32),
            pltpu.VMEM_SHARED((NP,), jnp.float32),
        ],
    )
    def k(src_h, dst_h, val_h, acc1_h, dv_h, par_h, acc_out,
          ta, tb, den, dva, dvb, pv2, src_v, dst_v, val_v,
          s0s_v, s1s_v, s0d_v, s1d_v, val0, val1, val2, val3, val4, val5,
          s0_sp, s1_sp, q0_sp, q1_sp, t00_sp, t01_sp, t10_sp, t11_sp):
        c = lax.axis_index("c")
        s = lax.axis_index("s")
        wid = c * 16 + s
        row0 = s * NSL
        ebase = wid * TEDGE
        f32 = jnp.float32

        _zero1d(ta, NSL)
        for ref in [q0_sp, q1_sp, t00_sp, t01_sp, t10_sp, t11_sp]:
            pltpu.sync_copy(ta, ref.at[pl.ds(row0, NSL)])

        for h, s_sp in ((0, s0_sp), (1, s1_sp)):
            pltpu.sync_copy(acc1_h.at[0, h, 0, pl.ds(row0, NSL)], ta)
            pltpu.sync_copy(acc1_h.at[1, h, 0, pl.ds(row0, NSL)], tb)

            def dbody(i, _):
                sl = pl.ds(i * 16, 16)
                den[sl] = ta[sl] + tb[sl] + f32(1e-9)
                return 0
            lax.fori_loop(0, NSL // 16, dbody, 0)

            pltpu.sync_copy(acc1_h.at[0, 2 + h, 0, pl.ds(row0, NSL)], ta)
            pltpu.sync_copy(acc1_h.at[1, 2 + h, 0, pl.ds(row0, NSL)], tb)

            def nbody(i, _):
                sl = pl.ds(i * 16, 16)
                ta[sl] = (ta[sl] + tb[sl]) / den[sl]
                return 0
            lax.fori_loop(0, NSL // 16, nbody, 0)
            pltpu.sync_copy(ta, s_sp.at[pl.ds(row0, NSL)])

        pltpu.sync_copy(par_h, pv2)
        pltpu.sync_copy(dv_h.at[0, 0, :], dva)
        pltpu.sync_copy(dv_h.at[1, 0, :], dvb)
        plsc.subcore_barrier()

        dmax = jnp.maximum(dva[...], dvb[...])
        pvv = pv2[...]
        a00 = _splat(pvv, 0); a01 = _splat(pvv, 1)
        a10 = _splat(pvv, 2); a11 = _splat(pvv, 3)
        b00 = _splat(pvv, 4); b01 = _splat(pvv, 5)
        b10 = _splat(pvv, 6); b11 = _splat(pvv, 7)
        em0 = dmax * (jnp.maximum(a00, 0.0) + jnp.maximum(a10, 0.0))
        em1 = dmax * (jnp.maximum(a01, 0.0) + jnp.maximum(a11, 0.0))

        def ebody(kk, _):
            b = ebase + kk * CHUNK
            pltpu.sync_copy(src_h.at[pl.ds(b, CHUNK)], src_v)
            pltpu.sync_copy(dst_h.at[pl.ds(b, CHUNK)], dst_v)
            pltpu.sync_copy(val_h.at[pl.ds(b, CHUNK)], val_v)
            pltpu.sync_copy(s0_sp.at[src_v], s0s_v)
            pltpu.sync_copy(s1_sp.at[src_v], s1s_v)
            pltpu.sync_copy(s0_sp.at[dst_v], s0d_v)
            pltpu.sync_copy(s1_sp.at[dst_v], s1d_v)

            def cbody(i, _):
                sl = pl.ds(i * 16, 16)
                vv = val_v[sl]
                s0s = s0s_v[sl]; s1s = s1s_v[sl]
                s0d = s0d_v[sl]; s1d = s1d_v[sl]
                er0 = s0d * b00 + s1d * b10
                ee0 = jnp.exp(_lk(s0s * a00 + s1s * a10 + er0) - _lk(er0 + em0)) * vv
                er1 = s0d * b01 + s1d * b11
                ee1 = jnp.exp(_lk(s0s * a01 + s1s * a11 + er1) - _lk(er1 + em1)) * vv
                val0[sl] = ee0
                val1[sl] = ee1
                val2[sl] = ee0 * s0s
                val3[sl] = ee0 * s1s
                val4[sl] = ee1 * s0s
                val5[sl] = ee1 * s1s
                return 0
            lax.fori_loop(0, CHUNK // 16, cbody, 0)
            pltpu.sync_copy(val0, q0_sp.at[dst_v], add=True)
            pltpu.sync_copy(val1, q1_sp.at[dst_v], add=True)
            pltpu.sync_copy(val2, t00_sp.at[dst_v], add=True)
            pltpu.sync_copy(val3, t01_sp.at[dst_v], add=True)
            pltpu.sync_copy(val4, t10_sp.at[dst_v], add=True)
            pltpu.sync_copy(val5, t11_sp.at[dst_v], add=True)
            return 0
        lax.fori_loop(0, NSUB, ebody, 0)
        plsc.subcore_barrier()

        for comp, ref in enumerate([q0_sp, q1_sp, t00_sp, t01_sp, t10_sp, t11_sp]):
            pltpu.sync_copy(ref.at[pl.ds(row0, NSL)], ta)
            pltpu.sync_copy(ta, acc_out.at[c, comp, 0, pl.ds(row0, NSL)])

    return k(srcp, dstp, valid, acc1, dvec, params2)


def _tc_epilogue(acc2, ut, w1t, b1, w2t, b2, w3t, b3, w4t, b4, w5t, b5):
    BLK = 2048
    NB = NP // BLK

    def body(a_ref, u_ref, w1, v1, w2, v2, w3, v3, w4, v4, w5, v5, o_ref):
        def blk(j, carry):
            sl = pl.ds(j * BLK, BLK)
            a = a_ref[0, :, sl] + a_ref[1, :, sl]
            d0 = a[0:1, :] + 1e-9
            d1 = a[1:2, :] + 1e-9
            t00 = a[2:3, :] / d0
            t01 = a[3:4, :] / d0
            t10 = a[4:5, :] / d1
            t11 = a[5:6, :] / d1
            T0 = jnp.concatenate([jnp.broadcast_to(t00, (HID, BLK)),
                                  jnp.broadcast_to(t10, (HID, BLK))], axis=0)
            T1 = jnp.concatenate([jnp.broadcast_to(t01, (HID, BLK)),
                                  jnp.broadcast_to(t11, (HID, BLK))], axis=0)
            O = jnp.maximum(T0 * u_ref[:, 0:1] + T1 * u_ref[:, 1:2], 0.0)
            return carry + jnp.sum(O, axis=1, keepdims=True)

        cs = lax.fori_loop(0, NB, blk, jnp.zeros((2 * HID, 1), jnp.float32))
        hg = (cs[:HID, :] + cs[HID:, :]) * (1.0 / (2.0 * N))
        x = jnp.maximum(jnp.dot(w1[...], hg, preferred_element_type=jnp.float32) + v1[...], 0.0)
        x = jnp.maximum(jnp.dot(w2[...], x, preferred_element_type=jnp.float32) + v2[...], 0.0)
        x = jnp.maximum(jnp.dot(w3[...], x, preferred_element_type=jnp.float32) + v3[...], 0.0)
        x = jnp.maximum(jnp.dot(w4[...], x, preferred_element_type=jnp.float32) + v4[...], 0.0)
        x = jnp.dot(w5[...], x, preferred_element_type=jnp.float32) + v5[...]
        ex = jnp.exp(x - jnp.max(x, axis=0, keepdims=True))
        o_ref[...] = ex / jnp.sum(ex, axis=0, keepdims=True)

    return pl.pallas_call(
        body,
        out_shape=jax.ShapeDtypeStruct((OUT, 1), jnp.float32),
    )(acc2, ut, w1t, b1, w2t, b2, w3t, b3, w4t, b4, w5t, b5)


def kernel(edge_index, W1, al1, ar1, W2, al2, ar2, cw1, cb1, cw2, cb2, cw3,
           cb3, cw4, cb4, cw5, cb5):
    W1r = W1.reshape(HEADS, HID)
    cl = (W1r * al1).sum(-1)
    cr = (W1r * ar1).sum(-1)
    params1 = jnp.zeros((16,), jnp.float32).at[0:2].set(cl).at[2:4].set(cr)

    W1p = jnp.maximum(W1.reshape(-1), 0.0)
    U = jnp.stack([W1p[c * HID:(c + 1) * HID] @ W2[c * HID:(c + 1) * HID, :]
                   for c in range(HEADS)])
    Ur = U.reshape(HEADS, HEADS, HID)
    A = (Ur * al2[None]).sum(-1)
    B = (Ur * ar2[None]).sum(-1)
    params2 = jnp.zeros((16,), jnp.float32).at[0:4].set(A.reshape(-1)).at[4:8].set(B.reshape(-1))

    padidx = (N + (jnp.arange(NPAD, dtype=jnp.int32) % (NP - N))).astype(jnp.int32)
    srcp = jnp.concatenate([edge_index[0].astype(jnp.int32), padidx])
    dstp = jnp.concatenate([edge_index[1].astype(jnp.int32), padidx])
    valid = jnp.concatenate([jnp.ones((E,), jnp.float32),
                             jnp.zeros((NPAD,), jnp.float32)])

    deg_p = _pass0(srcp, dstp, valid)
    acc1, dvec = _pass1(srcp, dstp, valid, deg_p, params1)
    acc2 = _pass2(srcp, dstp, valid, acc1, dvec, params2)

    out = _tc_epilogue(
        acc2.reshape(2, 6, NP), U.T,
        cw1.T, cb1.reshape(-1, 1), cw2.T, cb2.reshape(-1, 1),
        cw3.T, cb3.reshape(-1, 1), cw4.T, cb4.reshape(-1, 1),
        cw5.T, cb5.reshape(-1, 1))
    return out.reshape(1, OUT)


# speedup vs baseline: 656.5396x; 1.0769x over previous
import functools

import jax
import jax.numpy as jnp
from jax import lax
from jax.experimental import pallas as pl
from jax.experimental.pallas import tpu as pltpu
from jax.experimental.pallas import tpu_sc as plsc

N = 50000
E = 800000
HID = 32
HEADS = 2
OUT = 10

NP = 51200
NSL = NP // 16
EP = 819200
TEDGE = EP // 32
CHUNK = 6400
NSUB = TEDGE // CHUNK
NPAD = EP - E


def _lk(x):
    return jnp.where(x > 0, x, x * jnp.float32(0.2))


def _perm(v, idx):
    dn = lax.GatherDimensionNumbers(offset_dims=(), collapsed_slice_dims=(0,),
                                    start_index_map=(0,))
    return lax.gather(v, idx[:, None], dn, (1,),
                      mode=lax.GatherScatterMode.PROMISE_IN_BOUNDS)


def _splat(vec, j):
    return _perm(vec, jnp.full((16,), j, jnp.int32))


def _lane_max_splat(v):
    iota = lax.iota(jnp.int32, 16)
    for k in (8, 4, 2, 1):
        v = jnp.maximum(v, _perm(v, lax.bitwise_xor(iota, k)))
    return v


def _zero1d(ref, n):
    def body(i, _):
        ref[pl.ds(i * 16, 16)] = jnp.zeros((16,), jnp.float32)
        return 0
    lax.fori_loop(0, n // 16, body, 0)


def _sc_mesh():
    return plsc.VectorSubcoreMesh(core_axis_name="c", subcore_axis_name="s")


def _pass0(srcp, dstp, valid):
    @functools.partial(
        pl.kernel, mesh=_sc_mesh(),
        out_type=jax.ShapeDtypeStruct((2, 1, NP), jnp.float32),
        scratch_types=[
            pltpu.VMEM((NSL,), jnp.float32),
            pltpu.VMEM((CHUNK,), jnp.int32),
            pltpu.VMEM((CHUNK,), jnp.float32),
            pltpu.VMEM_SHARED((NP,), jnp.float32),
        ],
    )
    def k(src_h, dst_h, val_h, out_h, zt, dst_v, val_v, deg_sp):
        c = lax.axis_index("c")
        s = lax.axis_index("s")
        wid = c * 16 + s
        row0 = s * NSL
        ebase = wid * TEDGE

        _zero1d(zt, NSL)
        pltpu.sync_copy(zt, deg_sp.at[pl.ds(row0, NSL)])
        plsc.subcore_barrier()

        def ebody(kk, _):
            b = ebase + kk * CHUNK
            pltpu.sync_copy(dst_h.at[pl.ds(b, CHUNK)], dst_v)
            pltpu.sync_copy(val_h.at[pl.ds(b, CHUNK)], val_v)
            pltpu.sync_copy(val_v, deg_sp.at[dst_v], add=True)
            return 0
        lax.fori_loop(0, NSUB, ebody, 0)
        plsc.subcore_barrier()

        pltpu.sync_copy(deg_sp.at[pl.ds(row0, NSL)], zt)
        pltpu.sync_copy(zt, out_h.at[c, 0, pl.ds(row0, NSL)])

    return k(srcp, dstp, valid)


def _pass1(srcp, dstp, valid, deg_p, params1):
    @functools.partial(
        pl.kernel, mesh=_sc_mesh(),
        out_type=[jax.ShapeDtypeStruct((2, 4, 1, NP), jnp.float32),
                  jax.ShapeDtypeStruct((2, 1, 16), jnp.float32)],
        scratch_types=[
            pltpu.VMEM((NSL,), jnp.float32),
            pltpu.VMEM((NSL,), jnp.float32),
            pltpu.VMEM((16,), jnp.float32),
            pltpu.VMEM((16,), jnp.float32),
            pltpu.VMEM((256,), jnp.float32),
            pltpu.VMEM((CHUNK,), jnp.int32),
            pltpu.VMEM((CHUNK,), jnp.int32),
            pltpu.VMEM((CHUNK,), jnp.float32),
            pltpu.VMEM((CHUNK,), jnp.float32),
            pltpu.VMEM((CHUNK,), jnp.float32),
            pltpu.VMEM((CHUNK,), jnp.float32),
            pltpu.VMEM((CHUNK,), jnp.float32),
            pltpu.VMEM((CHUNK,), jnp.float32),
            pltpu.VMEM((CHUNK,), jnp.float32),
            pltpu.VMEM_SHARED((NP,), jnp.float32),
            pltpu.VMEM_SHARED((NP,), jnp.float32),
            pltpu.VMEM_SHARED((NP,), jnp.float32),
            pltpu.VMEM_SHARED((NP,), jnp.float32),
            pltpu.VMEM_SHARED((NP,), jnp.float32),
            pltpu.VMEM_SHARED((256,), jnp.float32),
        ],
    )
    def k(src_h, dst_h, val_h, deg_h, par_h, acc_out, dmax_out,
          ta, tb, pv, tmpf, sbuf, src_v, dst_v, val_v, degs_v, degd_v,
          val0, val1, val2, val3, deg_sp, d0_sp, d1_sp, n0_sp, n1_sp,
          stats_sp):
        c = lax.axis_index("c")
        s = lax.axis_index("s")
        wid = c * 16 + s
        row0 = s * NSL
        ebase = wid * TEDGE

        _zero1d(ta, NSL)
        pltpu.sync_copy(ta, d0_sp.at[pl.ds(row0, NSL)])
        pltpu.sync_copy(ta, d1_sp.at[pl.ds(row0, NSL)])
        pltpu.sync_copy(ta, n0_sp.at[pl.ds(row0, NSL)])
        pltpu.sync_copy(ta, n1_sp.at[pl.ds(row0, NSL)])

        pltpu.sync_copy(deg_h.at[0, 0, pl.ds(row0, NSL)], ta)
        pltpu.sync_copy(deg_h.at[1, 0, pl.ds(row0, NSL)], tb)

        def stg(i, m):
            v = ta[pl.ds(i * 16, 16)] + tb[pl.ds(i * 16, 16)]
            ta[pl.ds(i * 16, 16)] = v
            return jnp.maximum(m, v)
        mvec = lax.fori_loop(0, NSL // 16, stg, jnp.zeros((16,), jnp.float32))
        pltpu.sync_copy(ta, deg_sp.at[pl.ds(row0, NSL)])
        tmpf[...] = mvec
        pltpu.sync_copy(tmpf, stats_sp.at[pl.ds(s * 16, 16)])
        pltpu.sync_copy(par_h, pv)
        plsc.subcore_barrier()

        pltpu.sync_copy(stats_sp, sbuf)
        m = sbuf[pl.ds(0, 16)]
        for i in range(1, 16):
            m = jnp.maximum(m, sbuf[pl.ds(i * 16, 16)])
        dmax = _lane_max_splat(m)

        pvv = pv[...]
        cl0 = _splat(pvv, 0); cl1 = _splat(pvv, 1)
        cr0 = _splat(pvv, 2); cr1 = _splat(pvv, 3)
        em0 = jnp.maximum(cl0, 0.0) * dmax
        em1 = jnp.maximum(cl1, 0.0) * dmax

        def ebody(kk, _):
            b = ebase + kk * CHUNK
            pltpu.sync_copy(src_h.at[pl.ds(b, CHUNK)], src_v)
            pltpu.sync_copy(dst_h.at[pl.ds(b, CHUNK)], dst_v)
            pltpu.sync_copy(val_h.at[pl.ds(b, CHUNK)], val_v)
            pltpu.sync_copy(deg_sp.at[src_v], degs_v)
            pltpu.sync_copy(deg_sp.at[dst_v], degd_v)

            def cbody(i, _):
                sl = pl.ds(i * 16, 16)
                ds_ = degs_v[sl]; dd = degd_v[sl]; vv = val_v[sl]
                er = dd * cr0
                ee0 = jnp.exp(_lk(ds_ * cl0 + er) - _lk(er + em0)) * vv
                er1 = dd * cr1
                ee1 = jnp.exp(_lk(ds_ * cl1 + er1) - _lk(er1 + em1)) * vv
                val0[sl] = ee0
                val1[sl] = ee1
                val2[sl] = ee0 * ds_
                val3[sl] = ee1 * ds_
                return 0
            lax.fori_loop(0, CHUNK // 16, cbody, 0)
            pltpu.sync_copy(val0, d0_sp.at[dst_v], add=True)
            pltpu.sync_copy(val1, d1_sp.at[dst_v], add=True)
            pltpu.sync_copy(val2, n0_sp.at[dst_v], add=True)
            pltpu.sync_copy(val3, n1_sp.at[dst_v], add=True)
            return 0
        lax.fori_loop(0, NSUB, ebody, 0)
        plsc.subcore_barrier()

        for comp, ref in enumerate([d0_sp, d1_sp, n0_sp, n1_sp]):
            pltpu.sync_copy(ref.at[pl.ds(row0, NSL)], ta)
            pltpu.sync_copy(ta, acc_out.at[c, comp, 0, pl.ds(row0, NSL)])

        @pl.when(s == 0)
        def _():
            tmpf[...] = dmax
            pltpu.sync_copy(tmpf, dmax_out.at[c, 0, :])

    return k(srcp, dstp, valid, deg_p, params1)


def _pass2(srcp, dstp, valid, acc1, dvec, params2):
    @functools.partial(
        pl.kernel, mesh=_sc_mesh(),
        out_type=jax.ShapeDtypeStruct((2, 6, 1, NP), jnp.float32),
        scratch_types=[
            pltpu.VMEM((NSL,), jnp.float32),
            pltpu.VMEM((NSL,), jnp.float32),
            pltpu.VMEM((NSL,), jnp.float32),
            pltpu.VMEM((16,), jnp.float32),
            pltpu.VMEM((16,), jnp.float32),
            pltpu.VMEM((16,), jnp.float32),
            pltpu.VMEM((CHUNK,), jnp.int32),
            pltpu.VMEM((CHUNK,), jnp.int32),
            pltpu.VMEM((CHUNK,), jnp.float32),
            pltpu.VMEM((CHUNK,), jnp.float32),
            pltpu.VMEM((CHUNK,), jnp.float32),
            pltpu.VMEM((CHUNK,), jnp.float32),
            pltpu.VMEM((CHUNK,), jnp.float32),
            pltpu.VMEM((CHUNK,), jnp.float32),
            pltpu.VMEM((CHUNK,), jnp.float32),
            pltpu.VMEM((CHUNK,), jnp.float32),
            pltpu.VMEM((CHUNK,), jnp.float32),
            pltpu.VMEM((CHUNK,), jnp.float32),
            pltpu.VMEM((CHUNK,), jnp.float32),
            pltpu.VMEM_SHARED((NP,), jnp.float32),
            pltpu.VMEM_SHARED((NP,), jnp.float32),
            pltpu.VMEM_SHARED((NP,), jnp.float32),
            pltpu.VMEM_SHARED((NP,), jnp.float32),
            pltpu.VMEM_SHARED((NP,), jnp.float32),
            pltpu.VMEM_SHARED((NP,), jnp.float32),
            pltpu.VMEM_SHARED((NP,), jnp.float32),
            pltpu.VMEM_SHARED((NP,), jnp.float32),
        ],
    )
    def k(src_h, dst_h, val_h, acc1_h, dv_h, par_h, acc_out,
          ta, tb, den, dva, dvb, pv2, src_v, dst_v, val_v,
          s0s_v, s1s_v, s0d_v, s1d_v, val0, val1, val2, val3, val4, val5,
          s0_sp, s1_sp, q0_sp, q1_sp, t00_sp, t01_sp, t10_sp, t11_sp):
        c = lax.axis_index("c")
        s = lax.axis_index("s")
        wid = c * 16 + s
        row0 = s * NSL
        ebase = wid * TEDGE
        f32 = jnp.float32

        _zero1d(ta, NSL)
        for ref in [q0_sp, q1_sp, t00_sp, t01_sp, t10_sp, t11_sp]:
            pltpu.sync_copy(ta, ref.at[pl.ds(row0, NSL)])

        for h, s_sp in ((0, s0_sp), (1, s1_sp)):
            pltpu.sync_copy(acc1_h.at[0, h, 0, pl.ds(row0, NSL)], ta)
            pltpu.sync_copy(acc1_h.at[1, h, 0, pl.ds(row0, NSL)], tb)

            def dbody(i, _):
                sl = pl.ds(i * 16, 16)
                den[sl] = ta[sl] + tb[sl] + f32(1e-9)
                return 0
            lax.fori_loop(0, NSL // 16, dbody, 0)

            pltpu.sync_copy(acc1_h.at[0, 2 + h, 0, pl.ds(row0, NSL)], ta)
            pltpu.sync_copy(acc1_h.at[1, 2 + h, 0, pl.ds(row0, NSL)], tb)

            def nbody(i, _):
                sl = pl.ds(i * 16, 16)
                ta[sl] = (ta[sl] + tb[sl]) / den[sl]
                return 0
            lax.fori_loop(0, NSL // 16, nbody, 0)
            pltpu.sync_copy(ta, s_sp.at[pl.ds(row0, NSL)])

        pltpu.sync_copy(par_h, pv2)
        pltpu.sync_copy(dv_h.at[0, 0, :], dva)
        pltpu.sync_copy(dv_h.at[1, 0, :], dvb)
        plsc.subcore_barrier()

        dmax = jnp.maximum(dva[...], dvb[...])
        pvv = pv2[...]
        a00 = _splat(pvv, 0); a01 = _splat(pvv, 1)
        a10 = _splat(pvv, 2); a11 = _splat(pvv, 3)
        b00 = _splat(pvv, 4); b01 = _splat(pvv, 5)
        b10 = _splat(pvv, 6); b11 = _splat(pvv, 7)
        em0 = dmax * (jnp.maximum(a00, 0.0) + jnp.maximum(a10, 0.0))
        em1 = dmax * (jnp.maximum(a01, 0.0) + jnp.maximum(a11, 0.0))

        def ebody(kk, _):
            b = ebase + kk * CHUNK
            pltpu.sync_copy(src_h.at[pl.ds(b, CHUNK)], src_v)
            pltpu.sync_copy(dst_h.at[pl.ds(b, CHUNK)], dst_v)
            pltpu.sync_copy(val_h.at[pl.ds(b, CHUNK)], val_v)
            pltpu.sync_copy(s0_sp.at[src_v], s0s_v)
            pltpu.sync_copy(s1_sp.at[src_v], s1s_v)
            pltpu.sync_copy(s0_sp.at[dst_v], s0d_v)
            pltpu.sync_copy(s1_sp.at[dst_v], s1d_v)

            def cbody(i, _):
                sl = pl.ds(i * 16, 16)
                vv = val_v[sl]
                s0s = s0s_v[sl]; s1s = s1s_v[sl]
                s0d = s0d_v[sl]; s1d = s1d_v[sl]
                er0 = s0d * b00 + s1d * b10
                ee0 = jnp.exp(_lk(s0s * a00 + s1s * a10 + er0) - _lk(er0 + em0)) * vv
                er1 = s0d * b01 + s1d * b11
                ee1 = jnp.exp(_lk(s0s * a01 + s1s * a11 + er1) - _lk(er1 + em1)) * vv
                val0[sl] = ee0
                val1[sl] = ee1
                val2[sl] = ee0 * s0s
                val3[sl] = ee0 * s1s
                val4[sl] = ee1 * s0s
                val5[sl] = ee1 * s1s
                return 0
            lax.fori_loop(0, CHUNK // 16, cbody, 0)
            pltpu.sync_copy(val0, q0_sp.at[dst_v], add=True)
            pltpu.sync_copy(val1, q1_sp.at[dst_v], add=True)
            pltpu.sync_copy(val2, t00_sp.at[dst_v], add=True)
            pltpu.sync_copy(val3, t01_sp.at[dst_v], add=True)
            pltpu.sync_copy(val4, t10_sp.at[dst_v], add=True)
            pltpu.sync_copy(val5, t11_sp.at[dst_v], add=True)
            return 0
        lax.fori_loop(0, NSUB, ebody, 0)
        plsc.subcore_barrier()

        for comp, ref in enumerate([q0_sp, q1_sp, t00_sp, t01_sp, t10_sp, t11_sp]):
            pltpu.sync_copy(ref.at[pl.ds(row0, NSL)], ta)
            pltpu.sync_copy(ta, acc_out.at[c, comp, 0, pl.ds(row0, NSL)])

    return k(srcp, dstp, valid, acc1, dvec, params2)


def _tc_epilogue(acc2, ut, w1t, b1, w2t, b2, w3t, b3, w4t, b4, w5t, b5):
    BLK = 2048
    NB = NP // BLK

    def body(a_ref, u_ref, w1, v1, w2, v2, w3, v3, w4, v4, w5, v5, o_ref):
        def blk(j, carry):
            sl = pl.ds(j * BLK, BLK)
            a = a_ref[0, :, sl] + a_ref[1, :, sl]
            d0 = a[0:1, :] + 1e-9
            d1 = a[1:2, :] + 1e-9
            t00 = a[2:3, :] / d0
            t01 = a[3:4, :] / d0
            t10 = a[4:5, :] / d1
            t11 = a[5:6, :] / d1
            T0 = jnp.concatenate([jnp.broadcast_to(t00, (HID, BLK)),
                                  jnp.broadcast_to(t10, (HID, BLK))], axis=0)
            T1 = jnp.concatenate([jnp.broadcast_to(t01, (HID, BLK)),
                                  jnp.broadcast_to(t11, (HID, BLK))], axis=0)
            O = jnp.maximum(T0 * u_ref[:, 0:1] + T1 * u_ref[:, 1:2], 0.0)
            return carry + jnp.sum(O, axis=1, keepdims=True)

        cs = lax.fori_loop(0, NB, blk, jnp.zeros((2 * HID, 1), jnp.float32))
        hg = (cs[:HID, :] + cs[HID:, :]) * (1.0 / (2.0 * N))
        x = jnp.maximum(jnp.dot(w1[...], hg, preferred_element_type=jnp.float32) + v1[...], 0.0)
        x = jnp.maximum(jnp.dot(w2[...], x, preferred_element_type=jnp.float32) + v2[...], 0.0)
        x = jnp.maximum(jnp.dot(w3[...], x, preferred_element_type=jnp.float32) + v3[...], 0.0)
        x = jnp.maximum(jnp.dot(w4[...], x, preferred_element_type=jnp.float32) + v4[...], 0.0)
        x = jnp.dot(w5[...], x, preferred_element_type=jnp.float32) + v5[...]
        ex = jnp.exp(x - jnp.max(x, axis=0, keepdims=True))
        o_ref[...] = ex / jnp.sum(ex, axis=0, keepdims=True)

    return pl.pallas_call(
        body,
        out_shape=jax.ShapeDtypeStruct((OUT, 1), jnp.float32),
    )(acc2, ut, w1t, b1, w2t, b2, w3t, b3, w4t, b4, w5t, b5)


def kernel(edge_index, W1, al1, ar1, W2, al2, ar2, cw1, cb1, cw2, cb2, cw3,
           cb3, cw4, cb4, cw5, cb5):
    W1r = W1.reshape(HEADS, HID)
    cl = (W1r * al1).sum(-1)
    cr = (W1r * ar1).sum(-1)
    params1 = jnp.zeros((16,), jnp.float32).at[0:2].set(cl).at[2:4].set(cr)

    W1p = jnp.maximum(W1.reshape(-1), 0.0)
    U = jnp.stack([W1p[c * HID:(c + 1) * HID] @ W2[c * HID:(c + 1) * HID, :]
                   for c in range(HEADS)])
    Ur = U.reshape(HEADS, HEADS, HID)
    A = (Ur * al2[None]).sum(-1)
    B = (Ur * ar2[None]).sum(-1)
    params2 = jnp.zeros((16,), jnp.float32).at[0:4].set(A.reshape(-1)).at[4:8].set(B.reshape(-1))

    padidx = (N + (jnp.arange(NPAD, dtype=jnp.int32) % (NP - N))).astype(jnp.int32)
    srcp = jnp.concatenate([edge_index[0].astype(jnp.int32), padidx])
    dstp = jnp.concatenate([edge_index[1].astype(jnp.int32), padidx])
    valid = jnp.concatenate([jnp.ones((E,), jnp.float32),
                             jnp.zeros((NPAD,), jnp.float32)])

    deg_p = _pass0(srcp, dstp, valid)
    acc1, dvec = _pass1(srcp, dstp, valid, deg_p, params1)
    acc2 = _pass2(srcp, dstp, valid, acc1, dvec, params2)

    out = _tc_epilogue(
        acc2.reshape(2, 6, NP), U.T,
        cw1.T, cb1.reshape(-1, 1), cw2.T, cb2.reshape(-1, 1),
        cw3.T, cb3.reshape(-1, 1), cw4.T, cb4.reshape(-1, 1),
        cw5.T, cb5.reshape(-1, 1))
    return out.reshape(1, OUT)
